# baseline (speedup 1.0000x reference)
import jax
import jax.numpy as jnp
from jax.experimental import pallas as pl
from jax.experimental.pallas import tpu as pltpu

N_E = 1024
E_DIM = 64
NUM_SEG = 4
BETA = 0.25
HW = 1024
B = 16


def _split_bf16(v):
    hi = v.astype(jnp.bfloat16)
    lo = (v - hi.astype(jnp.float32)).astype(jnp.bfloat16)
    return hi, lo


def _vq_kernel(x_ref, w_ref, zq_ref, loss_ref):
    seg = pl.program_id(0)
    batch = pl.program_id(1)

    x = x_ref[0, 0]
    w = w_ref[0]

    m2 = jax.lax.dot_general(
        -2.0 * w, x, (((1,), (0,)), ((), ())),
        preferred_element_type=jnp.float32)
    w2 = jnp.sum(w * w, axis=1, keepdims=True)
    x2 = jnp.sum(x * x, axis=0, keepdims=True)
    scores = (x2 + w2) + m2

    smin = jnp.min(scores, axis=0, keepdims=True)
    mask = (scores == smin).astype(jnp.float32)

    wcat = jnp.concatenate(
        [w, jnp.ones((N_E, 8), jnp.float32)], axis=1)
    zq2 = jax.lax.dot_general(
        wcat, mask, (((0,), (0,)), ((), ())),
        preferred_element_type=jnp.float32)
    cnt = zq2[E_DIM:E_DIM + 1]
    scale = jnp.where(cnt <= 1.0, 1.0, 1.0 / cnt)
    zq = zq2[:E_DIM] * scale
    zq_ref[0, 0] = zq

    d = zq - x
    part = jnp.sum(d * d).reshape(1, 1, 1)

    @pl.when(batch == 0)
    def _():
        loss_ref[:, :, :] = jnp.zeros((1, 1, 1), jnp.float32)

    loss_ref[:, :, :] += part


@jax.jit
def kernel(z, emb):
    zr = z.reshape(B, NUM_SEG, E_DIM, HW)
    zq, loss = pl.pallas_call(
        _vq_kernel,
        grid=(NUM_SEG, B),
        in_specs=[
            pl.BlockSpec((1, 1, E_DIM, HW), lambda s, b: (b, s, 0, 0)),
            pl.BlockSpec((1, N_E, E_DIM), lambda s, b: (s, 0, 0)),
        ],
        out_specs=[
            pl.BlockSpec((1, 1, E_DIM, HW), lambda s, b: (b, s, 0, 0)),
            pl.BlockSpec((1, 1, 1), lambda s, b: (s, 0, 0)),
        ],
        out_shape=[
            jax.ShapeDtypeStruct((B, NUM_SEG, E_DIM, HW), jnp.float32),
            jax.ShapeDtypeStruct((NUM_SEG, 1, 1), jnp.float32),
        ],
        compiler_params=pltpu.CompilerParams(
            dimension_semantics=("parallel", "arbitrary")),
    )(zr, emb)
    total_loss = jnp.sum(loss) * ((1.0 + BETA) / (B * HW * E_DIM))
    return total_loss, zq.reshape(z.shape)


# speedup vs baseline: 1.0468x; 1.0468x over previous
import jax
import jax.numpy as jnp
from jax.experimental import pallas as pl
from jax.experimental.pallas import tpu as pltpu

N_E = 1024
E_DIM = 64
NUM_SEG = 4
BETA = 0.25
HW = 1024
B = 16


def _split_bf16(v):
    hi = v.astype(jnp.bfloat16)
    lo = (v - hi.astype(jnp.float32)).astype(jnp.bfloat16)
    return hi, lo


def _vq_kernel(x_ref, w_ref, zq_ref, loss_ref):
    seg = pl.program_id(0)
    batch = pl.program_id(1)

    x = x_ref[0].reshape(E_DIM, HW)
    w = w_ref[0]

    m2 = jax.lax.dot_general(
        -2.0 * w, x, (((1,), (0,)), ((), ())),
        preferred_element_type=jnp.float32)
    w2 = jnp.sum(w * w, axis=1, keepdims=True)
    x2 = jnp.sum(x * x, axis=0, keepdims=True)
    scores = (x2 + w2) + m2

    smin = jnp.min(scores, axis=0, keepdims=True)
    mask = (scores == smin).astype(jnp.float32)

    wcat = jnp.concatenate(
        [w, jnp.ones((N_E, 8), jnp.float32)], axis=1)
    zq2 = jax.lax.dot_general(
        wcat, mask, (((0,), (0,)), ((), ())),
        preferred_element_type=jnp.float32)
    cnt = zq2[E_DIM:E_DIM + 1]
    scale = jnp.where(cnt <= 1.0, 1.0, 1.0 / cnt)
    zq = zq2[:E_DIM] * scale
    zq_ref[0] = zq.reshape(E_DIM, 32, 32)

    d = zq - x
    part = jnp.sum(d * d).reshape(1, 1, 1)

    @pl.when(batch == 0)
    def _():
        loss_ref[:, :, :] = jnp.zeros((1, 1, 1), jnp.float32)

    loss_ref[:, :, :] += part


@jax.jit
def kernel(z, emb):
    zq, loss = pl.pallas_call(
        _vq_kernel,
        grid=(NUM_SEG, B),
        in_specs=[
            pl.BlockSpec((1, E_DIM, 32, 32), lambda s, b: (b, s, 0, 0)),
            pl.BlockSpec((1, N_E, E_DIM), lambda s, b: (s, 0, 0)),
        ],
        out_specs=[
            pl.BlockSpec((1, E_DIM, 32, 32), lambda s, b: (b, s, 0, 0)),
            pl.BlockSpec((1, 1, 1), lambda s, b: (s, 0, 0)),
        ],
        out_shape=[
            jax.ShapeDtypeStruct((B, 256, 32, 32), jnp.float32),
            jax.ShapeDtypeStruct((NUM_SEG, 1, 1), jnp.float32),
        ],
        compiler_params=pltpu.CompilerParams(
            dimension_semantics=("parallel", "arbitrary")),
    )(z, emb)
    total_loss = jnp.sum(loss) * ((1.0 + BETA) / (B * HW * E_DIM))
    return total_loss, zq


# speedup vs baseline: 1.9223x; 1.8362x over previous
import jax
import jax.numpy as jnp
from jax.experimental import pallas as pl
from jax.experimental.pallas import tpu as pltpu

N_E = 1024
E_DIM = 64
NUM_SEG = 4
BETA = 0.25
HW = 1024
B = 16
R = 512
N_ROWS = B * HW
N_CH = NUM_SEG * E_DIM


def _vq_kernel(x_ref, bd_ref, wc_ref, w2_ref, zq_ref, loss_ref):
    step = pl.program_id(0)

    x = x_ref[...]
    m2 = jax.lax.dot_general(
        x, bd_ref[...], (((1,), (0,)), ((), ())),
        preferred_element_type=jnp.float32)

    part = jnp.zeros((1, 1, 1), jnp.float32)
    outs = []
    for s in range(NUM_SEG):
        xs = x[:, s * E_DIM:(s + 1) * E_DIM]
        x2 = jnp.sum(xs * xs, axis=1, keepdims=True)
        bias = x2 + w2_ref[s]
        scores = bias + m2[:, s * N_E:(s + 1) * N_E]
        smin = jnp.min(scores, axis=1, keepdims=True)
        mask = (scores == smin).astype(jnp.float32)
        zq2 = jax.lax.dot_general(
            mask, wc_ref[s], (((1,), (0,)), ((), ())),
            preferred_element_type=jnp.float32)
        cnt = zq2[:, E_DIM:E_DIM + 1]
        scale = jnp.where(cnt <= 1.0, 1.0, 1.0 / cnt)
        zq_s = zq2[:, :E_DIM] * scale
        outs.append(zq_s)
        d = zq_s - xs
        part = part + jnp.sum(d * d).reshape(1, 1, 1)

    zq_ref[...] = jnp.concatenate(outs, axis=1)

    @pl.when(step == 0)
    def _():
        loss_ref[:, :, :] = jnp.zeros((1, 1, 1), jnp.float32)

    loss_ref[:, :, :] += part


@jax.jit
def kernel(z, emb):
    zt = jnp.transpose(z, (0, 2, 3, 1)).reshape(N_ROWS, N_CH)
    embT = jnp.transpose(emb, (0, 2, 1))

    bd = jnp.zeros((N_CH, NUM_SEG * N_E), jnp.float32)
    for s in range(NUM_SEG):
        bd = jax.lax.dynamic_update_slice(
            bd, -2.0 * embT[s], (s * E_DIM, s * N_E))
    wc = jnp.concatenate(
        [emb, jnp.ones((NUM_SEG, N_E, 8), jnp.float32)], axis=2)
    w2 = jnp.sum(emb * emb, axis=2).reshape(NUM_SEG, 1, N_E)

    grid = (N_ROWS // R,)
    zq, loss = pl.pallas_call(
        _vq_kernel,
        grid=grid,
        in_specs=[
            pl.BlockSpec((R, N_CH), lambda i: (i, 0)),
            pl.BlockSpec((N_CH, NUM_SEG * N_E), lambda i: (0, 0)),
            pl.BlockSpec((NUM_SEG, N_E, E_DIM + 8), lambda i: (0, 0, 0)),
            pl.BlockSpec((NUM_SEG, 1, N_E), lambda i: (0, 0, 0)),
        ],
        out_specs=[
            pl.BlockSpec((R, N_CH), lambda i: (i, 0)),
            pl.BlockSpec((1, 1, 1), lambda i: (0, 0, 0)),
        ],
        out_shape=[
            jax.ShapeDtypeStruct((N_ROWS, N_CH), jnp.float32),
            jax.ShapeDtypeStruct((1, 1, 1), jnp.float32),
        ],
        compiler_params=pltpu.CompilerParams(
            dimension_semantics=("arbitrary",)),
    )(zt, bd, wc, w2)

    total_loss = loss[0, 0, 0] * ((1.0 + BETA) / (N_ROWS * E_DIM))
    zq_out = jnp.transpose(zq.reshape(B, 32, 32, N_CH), (0, 3, 1, 2))
    return total_loss, zq_out


# speedup vs baseline: 1.9924x; 1.0365x over previous
import jax
import jax.numpy as jnp
from jax.experimental import pallas as pl
from jax.experimental.pallas import tpu as pltpu

N_E = 1024
E_DIM = 64
NUM_SEG = 4
BETA = 0.25
HW = 1024
B = 16
R = 1024
N_ROWS = B * HW
N_CH = NUM_SEG * E_DIM


def _vq_kernel(x_ref, bd_ref, wc_ref, w2_ref, zq_ref, loss_ref):
    step = pl.program_id(1)

    x = x_ref[...]
    m2 = jax.lax.dot_general(
        x, bd_ref[...], (((1,), (0,)), ((), ())),
        preferred_element_type=jnp.float32)

    part = jnp.zeros((1, 1, 1), jnp.float32)
    outs = []
    for s in range(NUM_SEG):
        xs = x[:, s * E_DIM:(s + 1) * E_DIM]
        x2 = jnp.sum(xs * xs, axis=1, keepdims=True)
        bias = x2 + w2_ref[s]
        scores = bias + m2[:, s * N_E:(s + 1) * N_E]
        smin = jnp.min(scores, axis=1, keepdims=True)
        mask = (scores == smin).astype(jnp.float32)
        zq2 = jax.lax.dot_general(
            mask, wc_ref[s], (((1,), (0,)), ((), ())),
            preferred_element_type=jnp.float32)
        cnt = zq2[:, E_DIM:E_DIM + 1]
        scale = jnp.where(cnt <= 1.0, 1.0, 1.0 / cnt)
        zq_s = zq2[:, :E_DIM] * scale
        outs.append(zq_s)
        d = zq_s - xs
        part = part + jnp.sum(d * d).reshape(1, 1, 1)

    zq_ref[...] = jnp.concatenate(outs, axis=1)

    @pl.when(step == 0)
    def _():
        loss_ref[:, :, :] = jnp.zeros((1, 1, 1), jnp.float32)

    loss_ref[:, :, :] += part


@jax.jit
def kernel(z, emb):
    zt = jnp.transpose(z, (0, 2, 3, 1)).reshape(N_ROWS, N_CH)
    embT = jnp.transpose(emb, (0, 2, 1))

    bd = jnp.zeros((N_CH, NUM_SEG * N_E), jnp.float32)
    for s in range(NUM_SEG):
        bd = jax.lax.dynamic_update_slice(
            bd, -2.0 * embT[s], (s * E_DIM, s * N_E))
    wc = jnp.concatenate(
        [emb, jnp.ones((NUM_SEG, N_E, 8), jnp.float32)], axis=2)
    w2 = jnp.sum(emb * emb, axis=2).reshape(NUM_SEG, 1, N_E)

    half = N_ROWS // R // 2
    zq, loss = pl.pallas_call(
        _vq_kernel,
        grid=(2, half),
        in_specs=[
            pl.BlockSpec((R, N_CH), lambda c, i: (c * half + i, 0)),
            pl.BlockSpec((N_CH, NUM_SEG * N_E), lambda c, i: (0, 0)),
            pl.BlockSpec((NUM_SEG, N_E, E_DIM + 8), lambda c, i: (0, 0, 0)),
            pl.BlockSpec((NUM_SEG, 1, N_E), lambda c, i: (0, 0, 0)),
        ],
        out_specs=[
            pl.BlockSpec((R, N_CH), lambda c, i: (c * half + i, 0)),
            pl.BlockSpec((1, 1, 1), lambda c, i: (c, 0, 0)),
        ],
        out_shape=[
            jax.ShapeDtypeStruct((N_ROWS, N_CH), jnp.float32),
            jax.ShapeDtypeStruct((2, 1, 1), jnp.float32),
        ],
        compiler_params=pltpu.CompilerParams(
            dimension_semantics=("parallel", "arbitrary")),
    )(zt, bd, wc, w2)

    total_loss = jnp.sum(loss) * ((1.0 + BETA) / (N_ROWS * E_DIM))
    zq_out = jnp.transpose(zq.reshape(B, 32, 32, N_CH), (0, 3, 1, 2))
    return total_loss, zq_out
